# baseline (speedup 1.0000x reference)
import dataclasses

import jax
import jax.numpy as jnp
from jax import lax
from jax.experimental import pallas as pl
from jax.experimental.pallas import tpu as pltpu
from jax.experimental.pallas import tpu_sc as plsc

N = 10000
E = 160000
D = 256
DH = 128

CH = 128
NCH = 1280
EPAD = NCH * CH
PADN = EPAD - E
NC = 2
NS = 16
ACC_ROWS = 10240
ROWS_PER_SUB = ACC_ROWS // NS

_mesh = plsc.VectorSubcoreMesh(core_axis_name="c", subcore_axis_name="s")


def _deg_kernel(dst_hbm, deg_hbm, didx, hist, tmp, acc640, exp, stage):
  c = lax.axis_index("c")
  s = lax.axis_index("s")
  w = c * NS + s
  Q = NCH // (NC * NS)
  ones16 = jnp.ones((16,), jnp.float32)

  @pl.loop(0, ACC_ROWS // 16)
  def _(i):
    hist[pl.ds(i * 16, 16)] = jnp.zeros((16,), jnp.float32)

  pltpu.sync_copy(dst_hbm.at[pl.ds(w * Q, Q)], didx)

  @pl.loop(0, Q)
  def _(j):
    @pl.loop(0, CH // 16)
    def _(k):
      iv = didx[j, pl.ds(k * 16, 16)]
      plsc.addupdate_scatter(hist, [iv], ones16)

  pltpu.sync_copy(hist, stage.at[s])
  plsc.subcore_barrier()

  pltpu.sync_copy(stage.at[0, pl.ds(s * ROWS_PER_SUB, ROWS_PER_SUB)], acc640)

  @pl.loop(1, NS)
  def _(hh):
    pltpu.sync_copy(stage.at[hh, pl.ds(s * ROWS_PER_SUB, ROWS_PER_SUB)], tmp)

    @pl.loop(0, ROWS_PER_SUB // 16)
    def _(v):
      acc640[pl.ds(v * 16, 16)] = (acc640[pl.ds(v * 16, 16)]
                                   + tmp[pl.ds(v * 16, 16)])

  iota16 = lax.iota(jnp.int32, 16)
  zeros16i = jnp.zeros((16,), jnp.int32)

  @pl.loop(0, ROWS_PER_SUB // CH)
  def _(r5):
    @pl.loop(0, CH // 16)
    def _(k):
      vals = acc640[pl.ds(r5 * CH + k * 16, 16)]
      plsc.store_scatter(exp, [iota16 + k * 16, zeros16i], vals)
    pltpu.sync_copy(exp, deg_hbm.at[c, pl.ds(s * ROWS_PER_SUB + r5 * CH, CH)])


def _scatter_kernel(src_hbm, dst_hbm, h2_hbm, out_hbm,
                    sidx, didx, rowsA, rowsB, acc,
                    semGA, semGB, semSA, semSB):
  c = lax.axis_index("c")
  s = lax.axis_index("s")
  P = NCH // NS

  @pl.loop(0, CH)
  def _(i):
    @pl.loop(0, DH // 16)
    def _(k):
      rowsA[i, pl.ds(k * 16, 16)] = jnp.zeros((16,), jnp.float32)

  @pl.loop(0, ROWS_PER_SUB // CH)
  def _(j):
    pltpu.sync_copy(rowsA, acc.at[pl.ds(s * ROWS_PER_SUB + j * CH, CH)])

  plsc.subcore_barrier()

  cc = 1 - c
  srcv = src_hbm
  dstv = dst_hbm
  h2c = h2_hbm.at[cc]

  def wait_g(buf, sem):
    pltpu.make_async_copy(h2c.at[sidx.at[0]], buf, sem).wait()

  def wait_s(buf, sem):
    pltpu.make_async_copy(buf, acc.at[didx.at[0]], sem).wait()

  P2 = P // 2

  @pl.loop(0, 2)
  def _(p):
    base = s * P + p * P2
    pltpu.sync_copy(srcv.at[pl.ds(base, P2)], sidx)
    pltpu.sync_copy(dstv.at[pl.ds(base, P2)], didx)
    pltpu.async_copy(h2c.at[sidx.at[0]], rowsA, semGA)
    pltpu.async_copy(h2c.at[sidx.at[1]], rowsB, semGB)

    @pl.loop(0, P2 // 2)
    def _(it):
      j0 = 2 * it
      wait_g(rowsA, semGA)
      pltpu.async_copy(rowsA, acc.at[didx.at[j0]], semSA, add=True)
      wait_g(rowsB, semGB)
      pltpu.async_copy(rowsB, acc.at[didx.at[j0 + 1]], semSB, add=True)

      @pl.when(j0 + 2 < P2)
      def _():
        wait_s(rowsA, semSA)
        pltpu.async_copy(h2c.at[sidx.at[j0 + 2]], rowsA, semGA)
        wait_s(rowsB, semSB)
        pltpu.async_copy(h2c.at[sidx.at[j0 + 3]], rowsB, semGB)

    wait_s(rowsA, semSA)
    wait_s(rowsB, semSB)

  plsc.subcore_barrier()
  pltpu.sync_copy(acc.at[pl.ds(s * ROWS_PER_SUB, ROWS_PER_SUB)],
                  out_hbm.at[cc, pl.ds(s * ROWS_PER_SUB, ROWS_PER_SUB)])


def _matmul_body(x_ref, w_ref, o_ref):
  o_ref[...] = jnp.dot(x_ref[...], w_ref[...],
                       preferred_element_type=jnp.float32)


def _scale_split_body(h_ref, d_ref, o_ref):
  deg = d_ref[0, :, :1] + d_ref[1, :, :1] + 1.0
  dinv = lax.rsqrt(deg)
  o_ref[0] = h_ref[:, :DH] * dinv
  o_ref[1] = h_ref[:, DH:] * dinv


def _finish_body(a_ref, h2_ref, d_ref, b_ref, o_ref):
  deg = d_ref[0, :, :1] + d_ref[1, :, :1] + 1.0
  dinv = lax.rsqrt(deg)
  o_ref[:, :DH] = (a_ref[0] + h2_ref[0]) * dinv + b_ref[0, :DH]
  o_ref[:, DH:] = (a_ref[1] + h2_ref[1]) * dinv + b_ref[0, DH:]


def kernel(x, edge_index, W, b):
  src = edge_index[0].astype(jnp.int32)
  dst = edge_index[1].astype(jnp.int32)
  src_p = jnp.concatenate([src, jnp.zeros((PADN,), jnp.int32)])
  dst_p = jnp.concatenate([dst, jnp.full((PADN,), N, jnp.int32)])
  src2d = src_p.reshape(NCH, CH)
  dst2d = dst_p.reshape(NCH, CH)

  cp = pltpu.CompilerParams()
  if "needs_layout_passes" in pltpu.CompilerParams.__dataclass_fields__:
    cp = dataclasses.replace(cp, needs_layout_passes=False)
  deg_fn = pl.kernel(
      _deg_kernel,
      out_type=jax.ShapeDtypeStruct((NC, ACC_ROWS, DH), jnp.float32),
      mesh=_mesh,
      compiler_params=cp,
      scratch_types=[
          pltpu.VMEM((NCH // (NC * NS), CH), jnp.int32),
          pltpu.VMEM((ACC_ROWS,), jnp.float32),
          pltpu.VMEM((ROWS_PER_SUB,), jnp.float32),
          pltpu.VMEM((ROWS_PER_SUB,), jnp.float32),
          pltpu.VMEM((CH, DH), jnp.float32),
          pltpu.VMEM_SHARED((NS, ACC_ROWS), jnp.float32),
      ],
  )
  degp = deg_fn(dst2d)

  RB = 2000
  h = pl.pallas_call(
      _matmul_body,
      grid=(N // RB,),
      in_specs=[pl.BlockSpec((RB, D), lambda i: (i, 0)),
                pl.BlockSpec((D, D), lambda i: (0, 0))],
      out_specs=pl.BlockSpec((RB, D), lambda i: (i, 0)),
      out_shape=jax.ShapeDtypeStruct((N, D), jnp.float32),
  )(x, W)

  h2 = pl.pallas_call(
      _scale_split_body,
      grid=(N // RB,),
      in_specs=[pl.BlockSpec((RB, D), lambda i: (i, 0)),
                pl.BlockSpec((NC, RB, DH), lambda i: (0, i, 0))],
      out_specs=pl.BlockSpec((NC, RB, DH), lambda i: (0, i, 0)),
      out_shape=jax.ShapeDtypeStruct((NC, N, DH), jnp.float32),
  )(h, degp)

  scat_fn = pl.kernel(
      _scatter_kernel,
      out_type=jax.ShapeDtypeStruct((NC, ACC_ROWS, DH), jnp.float32),
      mesh=_mesh,
      scratch_types=[
          pltpu.VMEM((NCH // NS // 2, CH), jnp.int32),
          pltpu.VMEM((NCH // NS // 2, CH), jnp.int32),
          pltpu.VMEM((CH, DH), jnp.float32),
          pltpu.VMEM((CH, DH), jnp.float32),
          pltpu.VMEM_SHARED((ACC_ROWS, DH), jnp.float32),
          pltpu.SemaphoreType.DMA,
          pltpu.SemaphoreType.DMA,
          pltpu.SemaphoreType.DMA,
          pltpu.SemaphoreType.DMA,
      ],
  )
  acc = scat_fn(src2d, dst2d, h2)

  b2 = b.reshape(1, D)
  out = pl.pallas_call(
      _finish_body,
      grid=(N // RB,),
      in_specs=[pl.BlockSpec((NC, RB, DH), lambda i: (0, i, 0)),
                pl.BlockSpec((NC, RB, DH), lambda i: (0, i, 0)),
                pl.BlockSpec((NC, RB, DH), lambda i: (0, i, 0)),
                pl.BlockSpec((1, D), lambda i: (0, 0))],
      out_specs=pl.BlockSpec((RB, D), lambda i: (i, 0)),
      out_shape=jax.ShapeDtypeStruct((N, D), jnp.float32),
  )(acc, h2, degp, b2)
  return out


# speedup vs baseline: 1.0403x; 1.0403x over previous
import dataclasses

import jax
import jax.numpy as jnp
from jax import lax
from jax.experimental import pallas as pl
from jax.experimental.pallas import tpu as pltpu
from jax.experimental.pallas import tpu_sc as plsc

N = 10000
E = 160000
D = 256
DH = 128
DW = 16

CH = 128
NCH = 1280
EPAD = NCH * CH
PADN = EPAD - E
NC = 2
NS = 16
ACC_ROWS = 10240
ROWS_PER_SUB = ACC_ROWS // NS

_mesh = plsc.VectorSubcoreMesh(core_axis_name="c", subcore_axis_name="s")


def _deg_kernel(dst_hbm, deg_hbm, didx, hist, tmp, acc640, exp, stage):
  c = lax.axis_index("c")
  s = lax.axis_index("s")
  w = c * NS + s
  Q = NCH // (NC * NS)
  ones16 = jnp.ones((16,), jnp.float32)

  @pl.loop(0, ACC_ROWS // 16)
  def _(i):
    hist[pl.ds(i * 16, 16)] = jnp.zeros((16,), jnp.float32)

  pltpu.sync_copy(dst_hbm.at[pl.ds(w * Q, Q)], didx)

  @pl.loop(0, Q)
  def _(j):
    @pl.loop(0, CH // 16)
    def _(k):
      iv = didx[j, pl.ds(k * 16, 16)]
      plsc.addupdate_scatter(hist, [iv], ones16)

  pltpu.sync_copy(hist, stage.at[s])
  plsc.subcore_barrier()

  pltpu.sync_copy(stage.at[0, pl.ds(s * ROWS_PER_SUB, ROWS_PER_SUB)], acc640)

  @pl.loop(1, NS)
  def _(hh):
    pltpu.sync_copy(stage.at[hh, pl.ds(s * ROWS_PER_SUB, ROWS_PER_SUB)], tmp)

    @pl.loop(0, ROWS_PER_SUB // 16)
    def _(v):
      acc640[pl.ds(v * 16, 16)] = (acc640[pl.ds(v * 16, 16)]
                                   + tmp[pl.ds(v * 16, 16)])

  iota16 = lax.iota(jnp.int32, 16)
  zeros16i = jnp.zeros((16,), jnp.int32)

  @pl.loop(0, ROWS_PER_SUB // CH)
  def _(r5):
    @pl.loop(0, CH // 16)
    def _(k):
      vals = acc640[pl.ds(r5 * CH + k * 16, 16)]
      plsc.store_scatter(exp, [iota16 + k * 16, zeros16i], vals)
    pltpu.sync_copy(exp, deg_hbm.at[c, pl.ds(s * ROWS_PER_SUB + r5 * CH, CH)])


def _scatter_kernel(src_hbm, dst_hbm, h2_hbm, out_hbm,
                    sidx, didx, rowsA, rowsB, acc,
                    semGA, semGB, semSA, semSB):
  c = lax.axis_index("c")
  s = lax.axis_index("s")
  P = NCH // NS

  @pl.loop(0, CH)
  def _(i):
    @pl.loop(0, DH // 16)
    def _(k):
      rowsA[i, pl.ds(k * 16, 16)] = jnp.zeros((16,), jnp.float32)

  @pl.loop(0, ROWS_PER_SUB // CH)
  def _(j):
    pltpu.sync_copy(rowsA, acc.at[pl.ds(s * ROWS_PER_SUB + j * CH, CH)])

  plsc.subcore_barrier()

  srcv = src_hbm
  dstv = dst_hbm
  h2c = h2_hbm.at[c]

  def wait_g(buf, sem):
    pltpu.make_async_copy(h2c.at[sidx.at[0]], buf, sem).wait()

  def wait_s(buf, sem):
    pltpu.make_async_copy(buf, acc.at[didx.at[0]], sem).wait()

  P2 = P // 2

  @pl.loop(0, 2)
  def _(p):
    base = s * P + p * P2
    pltpu.sync_copy(srcv.at[pl.ds(base, P2)], sidx)
    pltpu.sync_copy(dstv.at[pl.ds(base, P2)], didx)
    pltpu.async_copy(h2c.at[sidx.at[0]], rowsA, semGA)
    pltpu.async_copy(h2c.at[sidx.at[1]], rowsB, semGB)

    @pl.loop(0, P2 // 2)
    def _(it):
      j0 = 2 * it
      wait_g(rowsA, semGA)
      pltpu.async_copy(rowsA, acc.at[didx.at[j0]], semSA, add=True)
      wait_g(rowsB, semGB)
      pltpu.async_copy(rowsB, acc.at[didx.at[j0 + 1]], semSB, add=True)

      @pl.when(j0 + 2 < P2)
      def _():
        wait_s(rowsA, semSA)
        pltpu.async_copy(h2c.at[sidx.at[j0 + 2]], rowsA, semGA)
        wait_s(rowsB, semSB)
        pltpu.async_copy(h2c.at[sidx.at[j0 + 3]], rowsB, semGB)

    wait_s(rowsA, semSA)
    wait_s(rowsB, semSB)

  plsc.subcore_barrier()
  pltpu.sync_copy(acc.at[pl.ds(s * ROWS_PER_SUB, ROWS_PER_SUB)],
                  out_hbm.at[c, pl.ds(s * ROWS_PER_SUB, ROWS_PER_SUB)])


def _matmul_body(x_ref, w_ref, o_ref):
  o_ref[...] = jnp.dot(x_ref[...], w_ref[...],
                       preferred_element_type=jnp.float32)


def _scale_split_body(h_ref, d_ref, o_ref):
  deg = d_ref[0, :, :1] + d_ref[1, :, :1] + 1.0
  dinv = lax.rsqrt(deg)
  o_ref[0] = h_ref[:, :DH] * dinv
  o_ref[1] = h_ref[:, DH:] * dinv


def _finish_body(a_ref, h2_ref, d_ref, b_ref, o_ref):
  deg = d_ref[0, :, :1] + d_ref[1, :, :1] + 1.0
  dinv = lax.rsqrt(deg)
  o_ref[:, :DH] = (a_ref[0] + h2_ref[0]) * dinv + b_ref[0, :DH]
  o_ref[:, DH:] = (a_ref[1] + h2_ref[1]) * dinv + b_ref[0, DH:]


def kernel(x, edge_index, W, b):
  src = edge_index[0].astype(jnp.int32)
  dst = edge_index[1].astype(jnp.int32)
  src_p = jnp.concatenate([src, jnp.zeros((PADN,), jnp.int32)])
  dst_p = jnp.concatenate([dst, jnp.full((PADN,), N, jnp.int32)])
  src2d = src_p.reshape(NCH, CH)
  dst2d = dst_p.reshape(NCH, CH)

  cp = pltpu.CompilerParams()
  if "needs_layout_passes" in pltpu.CompilerParams.__dataclass_fields__:
    cp = dataclasses.replace(cp, needs_layout_passes=False)
  deg_fn = pl.kernel(
      _deg_kernel,
      out_type=jax.ShapeDtypeStruct((NC, ACC_ROWS, DW), jnp.float32),
      mesh=_mesh,
      compiler_params=cp,
      scratch_types=[
          pltpu.VMEM((NCH // (NC * NS), CH), jnp.int32),
          pltpu.VMEM((ACC_ROWS,), jnp.float32),
          pltpu.VMEM((ROWS_PER_SUB,), jnp.float32),
          pltpu.VMEM((ROWS_PER_SUB,), jnp.float32),
          pltpu.VMEM((CH, DW), jnp.float32),
          pltpu.VMEM_SHARED((NS, ACC_ROWS), jnp.float32),
      ],
  )
  degp = deg_fn(dst2d)

  RB = 2000
  h = pl.pallas_call(
      _matmul_body,
      grid=(N // RB,),
      in_specs=[pl.BlockSpec((RB, D), lambda i: (i, 0)),
                pl.BlockSpec((D, D), lambda i: (0, 0))],
      out_specs=pl.BlockSpec((RB, D), lambda i: (i, 0)),
      out_shape=jax.ShapeDtypeStruct((N, D), jnp.float32),
  )(x, W)

  h2 = pl.pallas_call(
      _scale_split_body,
      grid=(N // RB,),
      in_specs=[pl.BlockSpec((RB, D), lambda i: (i, 0)),
                pl.BlockSpec((NC, RB, DW), lambda i: (0, i, 0))],
      out_specs=pl.BlockSpec((NC, RB, DH), lambda i: (0, i, 0)),
      out_shape=jax.ShapeDtypeStruct((NC, N, DH), jnp.float32),
  )(h, degp)

  scat_fn = pl.kernel(
      _scatter_kernel,
      out_type=jax.ShapeDtypeStruct((NC, ACC_ROWS, DH), jnp.float32),
      mesh=_mesh,
      scratch_types=[
          pltpu.VMEM((NCH // NS // 2, CH), jnp.int32),
          pltpu.VMEM((NCH // NS // 2, CH), jnp.int32),
          pltpu.VMEM((CH, DH), jnp.float32),
          pltpu.VMEM((CH, DH), jnp.float32),
          pltpu.VMEM_SHARED((ACC_ROWS, DH), jnp.float32),
          pltpu.SemaphoreType.DMA,
          pltpu.SemaphoreType.DMA,
          pltpu.SemaphoreType.DMA,
          pltpu.SemaphoreType.DMA,
      ],
  )
  acc = scat_fn(src2d, dst2d, h2)

  b2 = b.reshape(1, D)
  out = pl.pallas_call(
      _finish_body,
      grid=(N // RB,),
      in_specs=[pl.BlockSpec((NC, RB, DH), lambda i: (0, i, 0)),
                pl.BlockSpec((NC, RB, DH), lambda i: (0, i, 0)),
                pl.BlockSpec((NC, RB, DW), lambda i: (0, i, 0)),
                pl.BlockSpec((1, D), lambda i: (0, 0))],
      out_specs=pl.BlockSpec((RB, D), lambda i: (i, 0)),
      out_shape=jax.ShapeDtypeStruct((N, D), jnp.float32),
  )(acc, h2, degp, b2)
  return out
